# baseline (speedup 1.0000x reference)
import jax
import jax.numpy as jnp
from jax.experimental import pallas as pl
from jax.experimental.pallas import tpu as pltpu

POOL_SIZE = 100000
POOL_DIM = 128
EPS = 1e-8
HALF_ROWS = 10000
NUM_BLOCKS = POOL_SIZE // (2 * HALF_ROWS)

_T_DIMS = (((1,), (1,)), ((), ()))
_N_DIMS = (((1,), (0,)), ((), ()))


def _half(m, x2b, ones2b, xnsq):
    mb = m.astype(jnp.bfloat16)
    dots = jax.lax.dot_general(x2b, mb, _T_DIMS,
                               preferred_element_type=jnp.float32)
    nsq = jax.lax.dot_general(ones2b, mb * mb, _T_DIMS,
                              preferred_element_type=jnp.float32)
    sims = dots * jax.lax.rsqrt(jnp.maximum(nsq, EPS * EPS) * xnsq)
    return jax.lax.dot_general(sims.astype(jnp.bfloat16), mb, _N_DIMS,
                               preferred_element_type=jnp.float32)


def _body(x_ref, ma_ref, mb_ref, out_ref, acc_ref):
    i = pl.program_id(0)
    x2 = x_ref[...]
    x2b = x2.astype(jnp.bfloat16)
    ones2b = jnp.ones((1, POOL_DIM), jnp.bfloat16)
    xnsq = jnp.maximum(jnp.sum(x2 * x2), EPS * EPS)

    partial = (_half(ma_ref[...], x2b, ones2b, xnsq)
               + _half(mb_ref[...], x2b, ones2b, xnsq))

    @pl.when(i == 0)
    def _():
        acc_ref[...] = jnp.zeros_like(acc_ref)

    acc_ref[...] += partial

    @pl.when(i == NUM_BLOCKS - 1)
    def _():
        acc = acc_ref[...]
        out_ref[...] = acc / jnp.max(jnp.abs(acc))


def kernel(x, mem):
    out = pl.pallas_call(
        _body,
        grid=(NUM_BLOCKS,),
        in_specs=[
            pl.BlockSpec((1, POOL_DIM), lambda i: (0, 0)),
            pl.BlockSpec((HALF_ROWS, POOL_DIM), lambda i: (2 * i, 0)),
            pl.BlockSpec((HALF_ROWS, POOL_DIM), lambda i: (2 * i + 1, 0)),
        ],
        out_specs=pl.BlockSpec((1, POOL_DIM), lambda i: (0, 0)),
        out_shape=jax.ShapeDtypeStruct((1, POOL_DIM), jnp.float32),
        scratch_shapes=[pltpu.VMEM((1, POOL_DIM), jnp.float32)],
    )(x.reshape(1, POOL_DIM), mem, mem)
    return out.reshape(POOL_DIM)


# speedup vs baseline: 1.0389x; 1.0389x over previous
import jax
import jax.numpy as jnp
from jax.experimental import pallas as pl
from jax.experimental.pallas import tpu as pltpu

POOL_SIZE = 100000
POOL_DIM = 128
EPS = 1e-8
BLOCK_ROWS = 20000
NUM_BLOCKS = POOL_SIZE // BLOCK_ROWS
_CHUNK_STARTS = tuple(range(0, BLOCK_ROWS - POOL_DIM, POOL_DIM)) + (BLOCK_ROWS - POOL_DIM,)
_OVERLAP = POOL_DIM - (BLOCK_ROWS - (BLOCK_ROWS // POOL_DIM) * POOL_DIM)

_T_DIMS = (((1,), (1,)), ((), ()))
_N_DIMS = (((1,), (0,)), ((), ()))


def _body(x_ref, mem_ref, out_ref, acc_ref):
    i = pl.program_id(0)
    x2 = x_ref[...]
    ones2 = jnp.ones((1, POOL_DIM), jnp.float32)
    xnsq = jnp.maximum(jnp.sum(x2 * x2), EPS * EPS)

    m = mem_ref[...]
    dots = jax.lax.dot_general(x2, m, _T_DIMS,
                               preferred_element_type=jnp.float32)
    nsq = jax.lax.dot_general(ones2, m * m, _T_DIMS,
                              preferred_element_type=jnp.float32)
    sims = dots * jax.lax.rsqrt(jnp.maximum(nsq, EPS * EPS) * xnsq)
    partial = jax.lax.dot_general(sims, m, _N_DIMS,
                                  preferred_element_type=jnp.float32)

    @pl.when(i == 0)
    def _():
        acc_ref[...] = jnp.zeros_like(acc_ref)

    acc_ref[...] += partial

    @pl.when(i == NUM_BLOCKS - 1)
    def _():
        acc = acc_ref[...]
        out_ref[...] = acc / jnp.max(jnp.abs(acc))


def kernel(x, mem):
    out = pl.pallas_call(
        _body,
        grid=(NUM_BLOCKS,),
        in_specs=[
            pl.BlockSpec((1, POOL_DIM), lambda i: (0, 0)),
            pl.BlockSpec((BLOCK_ROWS, POOL_DIM), lambda i: (i, 0)),
        ],
        out_specs=pl.BlockSpec((1, POOL_DIM), lambda i: (0, 0)),
        out_shape=jax.ShapeDtypeStruct((1, POOL_DIM), jnp.float32),
        scratch_shapes=[pltpu.VMEM((1, POOL_DIM), jnp.float32)],
    )(x.reshape(1, POOL_DIM), mem)
    return out.reshape(POOL_DIM)


# speedup vs baseline: 1.1423x; 1.0995x over previous
import jax
import jax.numpy as jnp
from jax.experimental import pallas as pl
from jax.experimental.pallas import tpu as pltpu

POOL_SIZE = 100000
POOL_DIM = 128
EPS = 1e-8
BLOCK_ROWS = 20000
NUM_BLOCKS = POOL_SIZE // BLOCK_ROWS
_CHUNK_STARTS = tuple(range(0, BLOCK_ROWS - POOL_DIM, POOL_DIM)) + (BLOCK_ROWS - POOL_DIM,)
_OVERLAP = POOL_DIM - (BLOCK_ROWS - (BLOCK_ROWS // POOL_DIM) * POOL_DIM)

_T_DIMS = (((1,), (1,)), ((), ()))
_N_DIMS = (((1,), (0,)), ((), ()))


def _body(x_ref, mem_ref, out_ref, acc_ref):
    i = pl.program_id(0)
    x2 = x_ref[...]
    xnsq = jnp.maximum(jnp.sum(x2 * x2), EPS * EPS)

    m = mem_ref[...]
    dots = jax.lax.dot_general(x2, m, _T_DIMS,
                               preferred_element_type=jnp.float32)
    nsq = jnp.sum(m * m, axis=1).reshape(1, BLOCK_ROWS)
    sims = dots * jax.lax.rsqrt(jnp.maximum(nsq, EPS * EPS) * xnsq)
    partial = jax.lax.dot_general(sims, m, _N_DIMS,
                                  preferred_element_type=jnp.float32)

    @pl.when(i == 0)
    def _():
        acc_ref[...] = jnp.zeros_like(acc_ref)

    acc_ref[...] += partial

    @pl.when(i == NUM_BLOCKS - 1)
    def _():
        acc = acc_ref[...]
        out_ref[...] = acc / jnp.max(jnp.abs(acc))


def kernel(x, mem):
    out = pl.pallas_call(
        _body,
        grid=(NUM_BLOCKS,),
        in_specs=[
            pl.BlockSpec((1, POOL_DIM), lambda i: (0, 0)),
            pl.BlockSpec((BLOCK_ROWS, POOL_DIM), lambda i: (i, 0)),
        ],
        out_specs=pl.BlockSpec((1, POOL_DIM), lambda i: (0, 0)),
        out_shape=jax.ShapeDtypeStruct((1, POOL_DIM), jnp.float32),
        scratch_shapes=[pltpu.VMEM((1, POOL_DIM), jnp.float32)],
    )(x.reshape(1, POOL_DIM), mem)
    return out.reshape(POOL_DIM)
